# baseline (speedup 1.0000x reference)
import functools

import jax
import jax.numpy as jnp
from jax import lax
from jax.experimental import pallas as pl
from jax.experimental.pallas import tpu as pltpu
from jax.experimental.pallas import tpu_sc as plsc

N = 10000
E = 320000
D = 128
NP = 10240
NC = 2
NS = 16
NW = NC * NS
CH = 128
NCHK = 80
EP = NW * CH * NCHK
RPT = NP // NS
BLK = 1024
EB = 4000

_f32 = jnp.float32
_bf16 = jnp.bfloat16
_i32 = jnp.int32


def _linear_body(x_ref, w_ref, b_ref, o_ref):
    o_ref[...] = jnp.dot(x_ref[...], w_ref[...],
                         preferred_element_type=_f32) + b_ref[...]


_linear = pl.pallas_call(
    _linear_body,
    grid=(NP // BLK,),
    in_specs=[
        pl.BlockSpec((BLK, D), lambda i: (i, 0)),
        pl.BlockSpec((D, D), lambda i: (0, 0)),
        pl.BlockSpec((1, D), lambda i: (0, 0)),
    ],
    out_specs=pl.BlockSpec((BLK, D), lambda i: (i, 0)),
    out_shape=jax.ShapeDtypeStruct((NP, D), _f32),
)


def _counts_body(dst_ref, o_ref):
    d = dst_ref[...]
    hrow = lax.broadcasted_iota(_i32, (1, NP // D), 1)
    col = lax.broadcasted_iota(_i32, (1, D), 1)
    u = (lax.shift_right_logical(d, 7) == hrow).astype(_bf16)
    v = ((d & 127) == col).astype(_bf16)
    blk_counts = lax.dot_general(u, v, (((0,), (0,)), ((), ())),
                                 preferred_element_type=_f32)

    @pl.when(pl.program_id(0) == 0)
    def _():
        o_ref[...] = jnp.zeros_like(o_ref)

    o_ref[...] += blk_counts


_counts = pl.pallas_call(
    _counts_body,
    grid=(E // EB,),
    in_specs=[pl.BlockSpec((EB, 1), lambda i: (i, 0))],
    out_specs=pl.BlockSpec((NP // D, D), lambda i: (0, 0)),
    out_shape=jax.ShapeDtypeStruct((NP // D, D), _f32),
)


def _merge_linear_body(acc_ref, cnt_ref, xt_ref, w_ref, b_ref, o_ref):
    s = acc_ref[0] + acc_ref[1]
    h = jnp.maximum(s / jnp.maximum(cnt_ref[...], 1.0), 0.0) + xt_ref[...]
    o_ref[...] = jnp.dot(h, w_ref[...], preferred_element_type=_f32) + b_ref[...]


_merge_linear = pl.pallas_call(
    _merge_linear_body,
    grid=(NP // BLK,),
    in_specs=[
        pl.BlockSpec((NC, BLK, D), lambda i: (0, i, 0)),
        pl.BlockSpec((BLK, 1), lambda i: (i, 0)),
        pl.BlockSpec((BLK, D), lambda i: (i, 0)),
        pl.BlockSpec((D, D), lambda i: (0, 0)),
        pl.BlockSpec((1, D), lambda i: (0, 0)),
    ],
    out_specs=pl.BlockSpec((BLK, D), lambda i: (i, 0)),
    out_shape=jax.ShapeDtypeStruct((NP, D), _f32),
)


def _merge_body(acc_ref, cnt_ref, xt_ref, o_ref):
    s = acc_ref[0] + acc_ref[1]
    o_ref[...] = jnp.maximum(s / jnp.maximum(cnt_ref[...], 1.0), 0.0) + xt_ref[...]


_merge = pl.pallas_call(
    _merge_body,
    grid=(NP // BLK,),
    in_specs=[
        pl.BlockSpec((NC, BLK, D), lambda i: (0, i, 0)),
        pl.BlockSpec((BLK, 1), lambda i: (i, 0)),
        pl.BlockSpec((BLK, D), lambda i: (i, 0)),
    ],
    out_specs=pl.BlockSpec((BLK, D), lambda i: (i, 0)),
    out_shape=jax.ShapeDtypeStruct((NP, D), _f32),
)


_mesh = plsc.VectorSubcoreMesh(core_axis_name="c", subcore_axis_name="s",
                               num_cores=NC, num_subcores=NS)


@functools.partial(
    pl.kernel,
    out_type=jax.ShapeDtypeStruct((NC, NP, D), _f32),
    mesh=_mesh,
    scratch_types=[
        pltpu.VMEM((NCHK, CH), _i32),
        pltpu.VMEM((NCHK, CH), _i32),
        pltpu.VMEM((CH, D), _f32),
        pltpu.VMEM_SHARED((NP, D), _f32),
    ],
)
def _sc_aggregate(xt_hbm, src_hbm, dst_hbm, zero_hbm, out_hbm,
                  sidx, didx, gbuf, acc):
    cid = lax.axis_index("c")
    sid = lax.axis_index("s")
    wid = sid * NC + cid
    rows = pl.ds(sid * RPT, RPT)
    pltpu.sync_copy(src_hbm.at[wid], sidx)
    pltpu.sync_copy(dst_hbm.at[wid], didx)
    pltpu.sync_copy(zero_hbm.at[rows], acc.at[rows])
    plsc.subcore_barrier()

    @pl.loop(0, NCHK)
    def _(j):
        pltpu.sync_copy(xt_hbm.at[sidx.at[j]], gbuf)
        pltpu.sync_copy(gbuf, acc.at[didx.at[j]], add=True)

    plsc.subcore_barrier()
    pltpu.sync_copy(acc.at[rows], out_hbm.at[cid, rows])


def kernel(x, edge_index, W0, b0, W1, b1):
    xp = jnp.zeros((NP, D), _f32).at[:N].set(x)
    src = edge_index[0].astype(_i32)
    dst = edge_index[1].astype(_i32)
    src3 = jnp.concatenate([src, jnp.zeros((EP - E,), _i32)]).reshape(NW, NCHK, CH)
    dst3 = jnp.concatenate([dst, jnp.full((EP - E,), NP - 1, _i32)]).reshape(NW, NCHK, CH)
    zeros_d = jnp.zeros((NP, D), _f32)
    w0t = W0.T
    w1t = W1.T
    b0r = b0.reshape(1, D)
    b1r = b1.reshape(1, D)

    cnt = _counts(dst.reshape(E, 1)).reshape(NP, 1)
    xt0 = _linear(xp, w0t, b0r)
    acc0 = _sc_aggregate(xt0, src3, dst3, zeros_d)
    xt1 = _merge_linear(acc0, cnt, xt0, w1t, b1r)
    acc1 = _sc_aggregate(xt1, src3, dst3, zeros_d)
    out = _merge(acc1, cnt, xt1)
    return out[:N]


# speedup vs baseline: 2.4320x; 2.4320x over previous
import functools

import jax
import jax.numpy as jnp
from jax import lax
from jax.experimental import pallas as pl
from jax.experimental.pallas import tpu as pltpu
from jax.experimental.pallas import tpu_sc as plsc

N = 10000
E = 320000
D = 128
NP = 10240
NC = 2
NS = 16
NW = NC * NS
CH = 128
NCHK = 80
EP = NW * CH * NCHK
RPT = NP // NS
BLK = 1024
EB = 4000

_f32 = jnp.float32
_bf16 = jnp.bfloat16
_i32 = jnp.int32


def _linear_body(x_ref, w_ref, b_ref, o_ref):
    o_ref[...] = jnp.dot(x_ref[...], w_ref[...],
                         preferred_element_type=_f32) + b_ref[...]


_linear = pl.pallas_call(
    _linear_body,
    grid=(NP // BLK,),
    in_specs=[
        pl.BlockSpec((BLK, D), lambda i: (i, 0)),
        pl.BlockSpec((D, D), lambda i: (0, 0)),
        pl.BlockSpec((1, D), lambda i: (0, 0)),
    ],
    out_specs=pl.BlockSpec((BLK, D), lambda i: (i, 0)),
    out_shape=jax.ShapeDtypeStruct((NP, D), _f32),
)


def _counts_body(dst_ref, o_ref):
    d = dst_ref[...]
    hrow = lax.broadcasted_iota(_i32, (1, NP // D), 1)
    col = lax.broadcasted_iota(_i32, (1, D), 1)
    u = (lax.shift_right_logical(d, 7) == hrow).astype(_bf16)
    v = ((d & 127) == col).astype(_bf16)
    blk_counts = lax.dot_general(u, v, (((0,), (0,)), ((), ())),
                                 preferred_element_type=_f32)

    @pl.when(pl.program_id(0) == 0)
    def _():
        o_ref[...] = jnp.zeros_like(o_ref)

    o_ref[...] += blk_counts


_counts = pl.pallas_call(
    _counts_body,
    grid=(E // EB,),
    in_specs=[pl.BlockSpec((EB, 1), lambda i: (i, 0))],
    out_specs=pl.BlockSpec((NP // D, D), lambda i: (0, 0)),
    out_shape=jax.ShapeDtypeStruct((NP // D, D), _f32),
)


def _merge_linear_body(acc_ref, cnt_ref, xt_ref, w_ref, b_ref, o_ref):
    s = acc_ref[0] + acc_ref[1]
    h = jnp.maximum(s / jnp.maximum(cnt_ref[...], 1.0), 0.0) + xt_ref[...]
    o_ref[...] = jnp.dot(h, w_ref[...], preferred_element_type=_f32) + b_ref[...]


_merge_linear = pl.pallas_call(
    _merge_linear_body,
    grid=(NP // BLK,),
    in_specs=[
        pl.BlockSpec((NC, BLK, D), lambda i: (0, i, 0)),
        pl.BlockSpec((BLK, 1), lambda i: (i, 0)),
        pl.BlockSpec((BLK, D), lambda i: (i, 0)),
        pl.BlockSpec((D, D), lambda i: (0, 0)),
        pl.BlockSpec((1, D), lambda i: (0, 0)),
    ],
    out_specs=pl.BlockSpec((BLK, D), lambda i: (i, 0)),
    out_shape=jax.ShapeDtypeStruct((NP, D), _f32),
)


def _merge_body(acc_ref, cnt_ref, xt_ref, o_ref):
    s = acc_ref[0] + acc_ref[1]
    o_ref[...] = jnp.maximum(s / jnp.maximum(cnt_ref[...], 1.0), 0.0) + xt_ref[...]


_merge = pl.pallas_call(
    _merge_body,
    grid=(NP // BLK,),
    in_specs=[
        pl.BlockSpec((NC, BLK, D), lambda i: (0, i, 0)),
        pl.BlockSpec((BLK, 1), lambda i: (i, 0)),
        pl.BlockSpec((BLK, D), lambda i: (i, 0)),
    ],
    out_specs=pl.BlockSpec((BLK, D), lambda i: (i, 0)),
    out_shape=jax.ShapeDtypeStruct((NP, D), _f32),
)


_mesh = plsc.VectorSubcoreMesh(core_axis_name="c", subcore_axis_name="s",
                               num_cores=NC, num_subcores=NS)


@functools.partial(
    pl.kernel,
    out_type=jax.ShapeDtypeStruct((NC, NP, D), _f32),
    mesh=_mesh,
    scratch_types=[
        pltpu.VMEM((NCHK, CH), _i32),
        pltpu.VMEM((NCHK, CH), _i32),
        pltpu.VMEM((CH, D), _f32),
        pltpu.VMEM_SHARED((NP, D), _f32),
    ],
)
def _sc_aggregate(xt_hbm, src_hbm, dst_hbm, zero_hbm, out_hbm,
                  sidx, didx, gbuf, acc):
    cid = lax.axis_index("c")
    sid = lax.axis_index("s")
    wid = sid * NC + cid
    rows = pl.ds(sid * RPT, RPT)
    pltpu.sync_copy(src_hbm.at[wid], sidx)
    pltpu.sync_copy(dst_hbm.at[wid], didx)
    pltpu.sync_copy(zero_hbm.at[rows], acc.at[rows])
    plsc.subcore_barrier()

    @pl.loop(0, NCHK)
    def _(j):
        pltpu.sync_copy(xt_hbm.at[sidx.at[j]], gbuf)
        pltpu.sync_copy(gbuf, acc.at[didx.at[j]], add=True)

    plsc.subcore_barrier()
    pltpu.sync_copy(acc.at[rows], out_hbm.at[cid, rows])


def kernel(x, edge_index, W0, b0, W1, b1):
    xp = jnp.zeros((NP, D), _f32).at[:N].set(x)
    src = edge_index[0].astype(_i32)
    dst = edge_index[1].astype(_i32)
    pad = jnp.arange(EP - E, dtype=_i32)
    src3 = jnp.concatenate([src, pad % N]).reshape(NW, NCHK, CH)
    dst3 = jnp.concatenate([dst, N + pad % (NP - N)]).reshape(NW, NCHK, CH)
    zeros_d = jnp.zeros((NP, D), _f32)
    w0t = W0.T
    w1t = W1.T
    b0r = b0.reshape(1, D)
    b1r = b1.reshape(1, D)

    cnt = _counts(dst.reshape(E, 1)).reshape(NP, 1)
    xt0 = _linear(xp, w0t, b0r)
    acc0 = _sc_aggregate(xt0, src3, dst3, zeros_d)
    xt1 = _merge_linear(acc0, cnt, xt0, w1t, b1r)
    acc1 = _sc_aggregate(xt1, src3, dst3, zeros_d)
    out = _merge(acc1, cnt, xt1)
    return out[:N]


# speedup vs baseline: 2.5179x; 1.0353x over previous
import functools

import jax
import jax.numpy as jnp
from jax import lax
from jax.experimental import pallas as pl
from jax.experimental.pallas import tpu as pltpu
from jax.experimental.pallas import tpu_sc as plsc

N = 10000
E = 320000
D = 128
NP = 10240
NC = 2
NS = 16
NW = NC * NS
CH = 128
NCHK = 80
HALF = NCHK // 2
EP = NW * CH * NCHK
RPT = NP // NS
BLK = 1024
EB = 4000

_f32 = jnp.float32
_bf16 = jnp.bfloat16
_i32 = jnp.int32


def _linear_body(x_ref, w_ref, b_ref, o_ref):
    o_ref[...] = jnp.dot(x_ref[...], w_ref[...],
                         preferred_element_type=_f32) + b_ref[...]


_linear = pl.pallas_call(
    _linear_body,
    grid=(NP // BLK,),
    in_specs=[
        pl.BlockSpec((BLK, D), lambda i: (i, 0)),
        pl.BlockSpec((D, D), lambda i: (0, 0)),
        pl.BlockSpec((1, D), lambda i: (0, 0)),
    ],
    out_specs=pl.BlockSpec((BLK, D), lambda i: (i, 0)),
    out_shape=jax.ShapeDtypeStruct((NP, D), _f32),
)


def _counts_body(dst_ref, o_ref):
    d = dst_ref[...]
    hrow = lax.broadcasted_iota(_i32, (1, NP // D), 1)
    col = lax.broadcasted_iota(_i32, (1, D), 1)
    u = (lax.shift_right_logical(d, 7) == hrow).astype(_bf16)
    v = ((d & 127) == col).astype(_bf16)
    blk_counts = lax.dot_general(u, v, (((0,), (0,)), ((), ())),
                                 preferred_element_type=_f32)

    @pl.when(pl.program_id(0) == 0)
    def _():
        o_ref[...] = jnp.zeros_like(o_ref)

    o_ref[...] += blk_counts


_counts = pl.pallas_call(
    _counts_body,
    grid=(E // EB,),
    in_specs=[pl.BlockSpec((EB, 1), lambda i: (i, 0))],
    out_specs=pl.BlockSpec((NP // D, D), lambda i: (0, 0)),
    out_shape=jax.ShapeDtypeStruct((NP // D, D), _f32),
)


def _merge_linear_body(acc_ref, cnt_ref, xt_ref, w_ref, b_ref, o_ref):
    s = acc_ref[0] + acc_ref[1]
    h = jnp.maximum(s / jnp.maximum(cnt_ref[...], 1.0), 0.0) + xt_ref[...]
    o_ref[...] = jnp.dot(h, w_ref[...], preferred_element_type=_f32) + b_ref[...]


_merge_linear = pl.pallas_call(
    _merge_linear_body,
    grid=(NP // BLK,),
    in_specs=[
        pl.BlockSpec((NC, BLK, D), lambda i: (0, i, 0)),
        pl.BlockSpec((BLK, 1), lambda i: (i, 0)),
        pl.BlockSpec((BLK, D), lambda i: (i, 0)),
        pl.BlockSpec((D, D), lambda i: (0, 0)),
        pl.BlockSpec((1, D), lambda i: (0, 0)),
    ],
    out_specs=pl.BlockSpec((BLK, D), lambda i: (i, 0)),
    out_shape=jax.ShapeDtypeStruct((NP, D), _f32),
)


def _merge_body(acc_ref, cnt_ref, xt_ref, o_ref):
    s = acc_ref[0] + acc_ref[1]
    o_ref[...] = jnp.maximum(s / jnp.maximum(cnt_ref[...], 1.0), 0.0) + xt_ref[...]


_merge = pl.pallas_call(
    _merge_body,
    grid=(NP // BLK,),
    in_specs=[
        pl.BlockSpec((NC, BLK, D), lambda i: (0, i, 0)),
        pl.BlockSpec((BLK, 1), lambda i: (i, 0)),
        pl.BlockSpec((BLK, D), lambda i: (i, 0)),
    ],
    out_specs=pl.BlockSpec((BLK, D), lambda i: (i, 0)),
    out_shape=jax.ShapeDtypeStruct((NP, D), _f32),
)


_mesh = plsc.VectorSubcoreMesh(core_axis_name="c", subcore_axis_name="s",
                               num_cores=NC, num_subcores=NS)


@functools.partial(
    pl.kernel,
    out_type=jax.ShapeDtypeStruct((NC, NP, D), _f32),
    mesh=_mesh,
    scratch_types=[
        pltpu.VMEM((HALF, CH), _i32),
        pltpu.VMEM((HALF, CH), _i32),
        pltpu.VMEM((2, CH, D), _f32),
        pltpu.VMEM_SHARED((NP, D), _f32),
        pltpu.SemaphoreType.DMA,
        pltpu.SemaphoreType.DMA,
    ],
)
def _sc_aggregate(xt_hbm, src_hbm, dst_hbm, zero_hbm, out_hbm,
                  sidx, didx, gbuf, acc, gsem0, gsem1):
    cid = lax.axis_index("c")
    sid = lax.axis_index("s")
    wid = sid * NC + cid
    rows = pl.ds(sid * RPT, RPT)
    pltpu.sync_copy(zero_hbm.at[rows], acc.at[rows])
    plsc.subcore_barrier()

    for h in range(2):
        pltpu.sync_copy(src_hbm.at[wid, h], sidx)
        pltpu.sync_copy(dst_hbm.at[wid, h], didx)

        @pl.loop(0, HALF, step=2)
        def _(j):
            g0 = pltpu.async_copy(xt_hbm.at[sidx.at[j]], gbuf.at[0], gsem0)
            g1 = pltpu.async_copy(xt_hbm.at[sidx.at[j + 1]], gbuf.at[1], gsem1)
            g0.wait()
            pltpu.sync_copy(gbuf.at[0], acc.at[didx.at[j]], add=True)
            g1.wait()
            pltpu.sync_copy(gbuf.at[1], acc.at[didx.at[j + 1]], add=True)

    plsc.subcore_barrier()
    pltpu.sync_copy(acc.at[rows], out_hbm.at[cid, rows])


def kernel(x, edge_index, W0, b0, W1, b1):
    xp = jnp.zeros((NP, D), _f32).at[:N].set(x)
    src = edge_index[0].astype(_i32)
    dst = edge_index[1].astype(_i32)
    pad = jnp.arange(EP - E, dtype=_i32)
    src3 = jnp.concatenate([src, pad % N]).reshape(NW, 2, HALF, CH)
    dst3 = jnp.concatenate([dst, N + pad % (NP - N)]).reshape(NW, 2, HALF, CH)
    zeros_d = jnp.zeros((NP, D), _f32)
    w0t = W0.T
    w1t = W1.T
    b0r = b0.reshape(1, D)
    b1r = b1.reshape(1, D)

    cnt = _counts(dst.reshape(E, 1)).reshape(NP, 1)
    xt0 = _linear(xp, w0t, b0r)
    acc0 = _sc_aggregate(xt0, src3, dst3, zeros_d)
    xt1 = _merge_linear(acc0, cnt, xt0, w1t, b1r)
    acc1 = _sc_aggregate(xt1, src3, dst3, zeros_d)
    out = _merge(acc1, cnt, xt1)
    return out[:N]
